# Initial kernel scaffold; baseline (speedup 1.0000x reference)
import jax
import jax.numpy as jnp
from jax.experimental import pallas as pl


def kernel(x):
    raise NotImplementedError("write your pallas kernel here")


# speedup vs baseline: 6.5607x; 6.5607x over previous
import functools

import jax
import jax.numpy as jnp
from jax import lax
from jax.experimental import pallas as pl
from jax.experimental.pallas import tpu as pltpu
from jax.experimental.pallas import tpu_sc as plsc

_R = 128
_N = 32768
_L = 16
_NC = 2
_NS = 16
_NW = _NC * _NS
_RPW = _R // _NW
_NV = _N // _L
_U = 8
_NB = 1024
_NG = _NB // _L
_K = (_N - 1) // 2


def _scan_select(hist, cums, kk):
    zeros_i = jnp.zeros((_L,), jnp.int32)

    def pa(g, tot):
        h = hist[pl.ds(g * _L, _L)]
        cs = plsc.cumsum(h) + tot
        cums[pl.ds(g * _L, _L)] = cs
        hist[pl.ds(g * _L, _L)] = zeros_i
        return jnp.max(cs)

    lax.fori_loop(0, _NG, pa, jnp.int32(0))

    iota = lax.iota(jnp.int32, _L)
    kt = kk + 1
    c1 = plsc.load_gather(cums, [iota * 64 + 63])
    g1 = plsc.all_reduce_ffs(c1 >= kt)
    c2 = plsc.load_gather(cums, [g1 * 64 + iota * 4 + 3])
    g2 = plsc.all_reduce_ffs(c2 >= kt)
    idx3 = jnp.minimum(g1 * 64 + g2 * 4 + iota, _NB - 1)
    c3 = plsc.load_gather(cums, [idx3])
    f3 = plsc.all_reduce_ffs(c3 >= kt)
    b = g1 * 64 + g2 * 4 + f3
    cb = plsc.load_gather(cums, [jnp.maximum(b - 1, 0)])
    cum_before = jnp.where(b >= 1, cb, 0)
    return b, kk - cum_before


def _row_median_mean(row_v, hist, cums):
    ones_i = jnp.ones((_L,), jnp.int32)

    def p1(i, acc):
        base = i * (_L * _U)
        for u in range(_U):
            v = row_v[pl.ds(base + u * _L, _L)]
            d1 = (v * 1024.0).astype(jnp.int32)
            plsc.addupdate_scatter(hist, [d1], ones_i)
            acc = acc + v
        return acc

    acc = lax.fori_loop(0, _NV // _U, p1, jnp.zeros((_L,), jnp.float32))
    mean = jnp.sum(acc) * (1.0 / _N)

    kk0 = jnp.full((_L,), _K, jnp.int32)
    b1, kk1 = _scan_select(hist, cums, kk0)

    def p2(i, carry):
        base = i * (_L * _U)
        for u in range(_U):
            v = row_v[pl.ds(base + u * _L, _L)]
            t = (v * 1048576.0).astype(jnp.int32)
            m = (t >> 10) == b1
            plsc.addupdate_scatter(hist, [t & (_NB - 1)], ones_i, mask=m)
        return carry

    lax.fori_loop(0, _NV // _U, p2, jnp.int32(0))
    b2, kk2 = _scan_select(hist, cums, kk1)

    pref = (b1 << 10) | b2

    def p3(i, carry):
        base = i * (_L * _U)
        for u in range(_U):
            v = row_v[pl.ds(base + u * _L, _L)]
            t = (v * 1073741824.0).astype(jnp.int32)
            m = (t >> 10) == pref
            plsc.addupdate_scatter(hist, [t & (_NB - 1)], ones_i, mask=m)
        return carry

    lax.fori_loop(0, _NV // _U, p3, jnp.int32(0))
    b3, _ = _scan_select(hist, cums, kk2)

    m_key = (b1 << 20) | (b2 << 10) | b3
    m_bit = m_key.astype(jnp.float32) * (2.0 ** -30)
    return m_bit, mean


def _sc_body(x_hbm, out_hbm, row_v, hist, cums, out_v):
    wid = lax.axis_index("s") * _NC + lax.axis_index("c")
    iota = lax.iota(jnp.int32, _L)

    zeros_i = jnp.zeros((_L,), jnp.int32)

    def z(g, c):
        hist[pl.ds(g * _L, _L)] = zeros_i
        return c

    lax.fori_loop(0, _NG, z, jnp.int32(0))

    out_acc = jnp.zeros((_L,), jnp.float32)
    for j in range(_RPW):
        row = wid * _RPW + j
        pltpu.sync_copy(x_hbm.at[row], row_v)
        m_bit, mean = _row_median_mean(row_v, hist, cums)
        margin = jnp.abs(m_bit - 0.5)
        delta = mean * margin
        rep = jnp.where(m_bit > 0.5, 0.5 + delta, m_bit + delta)
        out_acc = jnp.where(iota == j, rep, out_acc)

    out_v[...] = out_acc
    pltpu.sync_copy(out_v, out_hbm.at[wid])


@functools.cache
def _build():
    mesh = plsc.VectorSubcoreMesh(core_axis_name="c", subcore_axis_name="s")
    return functools.partial(
        pl.kernel,
        out_type=jax.ShapeDtypeStruct((_NW, _L), jnp.float32),
        mesh=mesh,
        scratch_types=[
            pltpu.VMEM((_N,), jnp.float32),
            pltpu.VMEM((_NB,), jnp.int32),
            pltpu.VMEM((_NB,), jnp.int32),
            pltpu.VMEM((_L,), jnp.float32),
        ],
        compiler_params=pltpu.CompilerParams(needs_layout_passes=False),
    )(_sc_body)


@jax.jit
def kernel(x):
    out2d = _build()(x)
    return out2d[:, :_RPW].reshape(-1)


# speedup vs baseline: 25.6614x; 3.9114x over previous
import functools

import jax
import jax.numpy as jnp
from jax import lax
from jax.experimental import pallas as pl
from jax.experimental.pallas import tpu as pltpu
from jax.experimental.pallas import tpu_sc as plsc

_R = 128
_N = 32768
_L = 16
_NC = 2
_NS = 16
_NW = _NC * _NS
_RPW = _R // _NW
_NV = _N // _L
_U = 8
_NB = 1024
_NG = _NB // _L
_K = (_N - 1) // 2


def _scan_select(hist, cums, kk):
    zeros_i = jnp.zeros((_L,), jnp.int32)

    def pa(g, tot):
        h = hist[pl.ds(g * _L, _L)]
        hist[pl.ds(g * _L, _L)] = zeros_i
        for u in range(1, _U):
            h = h + hist[pl.ds(u * _NB + g * _L, _L)]
            hist[pl.ds(u * _NB + g * _L, _L)] = zeros_i
        cs = plsc.cumsum(h) + tot
        cums[pl.ds(g * _L, _L)] = cs
        return jnp.max(cs)

    lax.fori_loop(0, _NG, pa, jnp.int32(0))

    iota = lax.iota(jnp.int32, _L)
    kt = kk + 1
    c1 = plsc.load_gather(cums, [iota * 64 + 63])
    g1 = plsc.all_reduce_ffs(c1 >= kt)
    c2 = plsc.load_gather(cums, [g1 * 64 + iota * 4 + 3])
    g2 = plsc.all_reduce_ffs(c2 >= kt)
    idx3 = jnp.minimum(g1 * 64 + g2 * 4 + iota, _NB - 1)
    c3 = plsc.load_gather(cums, [idx3])
    f3 = plsc.all_reduce_ffs(c3 >= kt)
    b = g1 * 64 + g2 * 4 + f3
    cb = plsc.load_gather(cums, [jnp.maximum(b - 1, 0)])
    cum_before = jnp.where(b >= 1, cb, 0)
    return b, kk - cum_before


def _row_median_mean(row_v, hist, cums):
    ones_i = jnp.ones((_L,), jnp.int32)

    @plsc.parallel_loop(0, _NV, unroll=_U, carry=jnp.zeros((_L,), jnp.float32))
    def p1(i, acc):
        v = row_v[pl.ds(i * _L, _L)]
        d1 = (v * 1024.0).astype(jnp.int32)
        plsc.addupdate_scatter(hist, [((i & (_U - 1)) << 10) + d1], ones_i)
        return acc + v

    mean = jnp.sum(p1) * (1.0 / _N)

    kk0 = jnp.full((_L,), _K, jnp.int32)
    b1, kk1 = _scan_select(hist, cums, kk0)

    @plsc.parallel_loop(0, _NV, unroll=_U)
    def p2(i):
        v = row_v[pl.ds(i * _L, _L)]
        t = (v * 1048576.0).astype(jnp.int32)
        m = (t >> 10) == b1
        plsc.addupdate_scatter(
            hist, [((i & (_U - 1)) << 10) + (t & (_NB - 1))], ones_i, mask=m)

    b2, kk2 = _scan_select(hist, cums, kk1)

    pref = (b1 << 10) | b2

    @plsc.parallel_loop(0, _NV, unroll=_U)
    def p3(i):
        v = row_v[pl.ds(i * _L, _L)]
        t = (v * 1073741824.0).astype(jnp.int32)
        m = (t >> 10) == pref
        plsc.addupdate_scatter(
            hist, [((i & (_U - 1)) << 10) + (t & (_NB - 1))], ones_i, mask=m)

    b3, _ = _scan_select(hist, cums, kk2)

    m_key = (b1 << 20) | (b2 << 10) | b3
    m_bit = m_key.astype(jnp.float32) * (2.0 ** -30)
    return m_bit, mean


def _sc_body(x_hbm, out_hbm, row_a, row_b, hist, cums, out_v, sem_a, sem_b):
    wid = lax.axis_index("s") * _NC + lax.axis_index("c")
    iota = lax.iota(jnp.int32, _L)

    zeros_i = jnp.zeros((_L,), jnp.int32)

    @plsc.parallel_loop(0, _U * _NG, unroll=8)
    def z(g):
        hist[pl.ds(g * _L, _L)] = zeros_i

    rows = [row_a, row_b]
    sems = [sem_a, sem_b]
    base = wid * _RPW
    copies = [None, None]
    copies[0] = pltpu.async_copy(x_hbm.at[base], row_a, sem_a)

    out_acc = jnp.zeros((_L,), jnp.float32)
    for j in range(_RPW):
        copies[j % 2].wait()
        if j + 1 < _RPW:
            copies[(j + 1) % 2] = pltpu.async_copy(
                x_hbm.at[base + j + 1], rows[(j + 1) % 2], sems[(j + 1) % 2])
        m_bit, mean = _row_median_mean(rows[j % 2], hist, cums)
        margin = jnp.abs(m_bit - 0.5)
        delta = mean * margin
        rep = jnp.where(m_bit > 0.5, 0.5 + delta, m_bit + delta)
        out_acc = jnp.where(iota == j, rep, out_acc)

    out_v[...] = out_acc
    pltpu.sync_copy(out_v, out_hbm.at[wid])


@functools.cache
def _build():
    mesh = plsc.VectorSubcoreMesh(core_axis_name="c", subcore_axis_name="s")
    return functools.partial(
        pl.kernel,
        out_type=jax.ShapeDtypeStruct((_NW, _L), jnp.float32),
        mesh=mesh,
        scratch_types=[
            pltpu.VMEM((_N,), jnp.float32),
            pltpu.VMEM((_N,), jnp.float32),
            pltpu.VMEM((_U * _NB,), jnp.int32),
            pltpu.VMEM((_NB,), jnp.int32),
            pltpu.VMEM((_L,), jnp.float32),
            pltpu.SemaphoreType.DMA,
            pltpu.SemaphoreType.DMA,
        ],
        compiler_params=pltpu.CompilerParams(needs_layout_passes=False),
    )(_sc_body)


@jax.jit
def kernel(x):
    out2d = _build()(x)
    return out2d[:, :_RPW].reshape(-1)


# speedup vs baseline: 29.0637x; 1.1326x over previous
import functools

import jax
import jax.numpy as jnp
from jax import lax
from jax.experimental import pallas as pl
from jax.experimental.pallas import tpu as pltpu
from jax.experimental.pallas import tpu_sc as plsc

_R = 128
_N = 32768
_L = 16
_NC = 2
_NS = 16
_NW = _NC * _NS
_RPW = _R // _NW
_NV = _N // _L
_U = 8
_B = 11
_NB = 1 << _B
_NG = _NB // _L
_K = (_N - 1) // 2


def _scan_select(hist, cums, kk):
    zeros_i = jnp.zeros((_L,), jnp.int32)

    def pa(g, tot):
        h = hist[pl.ds(g * _L, _L)]
        hist[pl.ds(g * _L, _L)] = zeros_i
        for u in range(1, _U):
            h = h + hist[pl.ds(u * _NB + g * _L, _L)]
            hist[pl.ds(u * _NB + g * _L, _L)] = zeros_i
        cs = plsc.cumsum(h) + tot
        cums[pl.ds(g * _L, _L)] = cs
        return jnp.max(cs)

    lax.fori_loop(0, _NG, pa, jnp.int32(0))

    iota = lax.iota(jnp.int32, _L)
    kt = kk + 1
    c1 = plsc.load_gather(cums, [iota * 128 + 127])
    g1 = plsc.all_reduce_ffs(c1 >= kt)
    c2 = plsc.load_gather(cums, [g1 * 128 + iota * 8 + 7])
    g2 = plsc.all_reduce_ffs(c2 >= kt)
    idx3 = jnp.minimum(g1 * 128 + g2 * 8 + iota, _NB - 1)
    c3 = plsc.load_gather(cums, [idx3])
    f3 = plsc.all_reduce_ffs(c3 >= kt)
    b = g1 * 128 + g2 * 8 + f3
    cb = plsc.load_gather(cums, [jnp.maximum(b - 1, 0)])
    cum_before = jnp.where(b >= 1, cb, 0)
    return b, kk - cum_before


def _row_median_mean(row_v, hist, cums):
    ones_i = jnp.ones((_L,), jnp.int32)

    @plsc.parallel_loop(0, _NV, unroll=_U, carry=jnp.zeros((_L,), jnp.float32))
    def p1(i, acc):
        v = row_v[pl.ds(i * _L, _L)]
        d1 = (v * float(_NB)).astype(jnp.int32)
        plsc.addupdate_scatter(hist, [((i & (_U - 1)) << _B) + d1], ones_i)
        return acc + v

    mean = jnp.sum(p1) * (1.0 / _N)

    kk0 = jnp.full((_L,), _K, jnp.int32)
    b1, kk1 = _scan_select(hist, cums, kk0)

    @plsc.parallel_loop(0, _NV, unroll=_U)
    def p2(i):
        v = row_v[pl.ds(i * _L, _L)]
        t = (v * float(_NB * _NB)).astype(jnp.int32)
        m = (t >> _B) == b1
        plsc.addupdate_scatter(
            hist, [((i & (_U - 1)) << _B) + (t & (_NB - 1))], ones_i, mask=m)

    b2, _ = _scan_select(hist, cums, kk1)

    m_key = (b1 << _B) | b2
    m_bit = (m_key.astype(jnp.float32) + 0.5) * (1.0 / (_NB * _NB))
    return m_bit, mean


def _sc_body(x_hbm, out_hbm, row_a, row_b, hist, cums, out_v, sem_a, sem_b):
    wid = lax.axis_index("s") * _NC + lax.axis_index("c")
    iota = lax.iota(jnp.int32, _L)

    zeros_i = jnp.zeros((_L,), jnp.int32)

    @plsc.parallel_loop(0, _U * _NG, unroll=8)
    def z(g):
        hist[pl.ds(g * _L, _L)] = zeros_i

    rows = [row_a, row_b]
    sems = [sem_a, sem_b]
    base = wid * _RPW
    copies = [None, None]
    copies[0] = pltpu.async_copy(x_hbm.at[base], row_a, sem_a)

    out_acc = jnp.zeros((_L,), jnp.float32)
    for j in range(_RPW):
        copies[j % 2].wait()
        if j + 1 < _RPW:
            copies[(j + 1) % 2] = pltpu.async_copy(
                x_hbm.at[base + j + 1], rows[(j + 1) % 2], sems[(j + 1) % 2])
        m_bit, mean = _row_median_mean(rows[j % 2], hist, cums)
        margin = jnp.abs(m_bit - 0.5)
        delta = mean * margin
        rep = jnp.where(m_bit > 0.5, 0.5 + delta, m_bit + delta)
        out_acc = jnp.where(iota == j, rep, out_acc)

    out_v[...] = out_acc
    pltpu.sync_copy(out_v, out_hbm.at[wid])


@functools.cache
def _build():
    mesh = plsc.VectorSubcoreMesh(core_axis_name="c", subcore_axis_name="s")
    return functools.partial(
        pl.kernel,
        out_type=jax.ShapeDtypeStruct((_NW, _L), jnp.float32),
        mesh=mesh,
        scratch_types=[
            pltpu.VMEM((_N,), jnp.float32),
            pltpu.VMEM((_N,), jnp.float32),
            pltpu.VMEM((_U * _NB,), jnp.int32),
            pltpu.VMEM((_NB,), jnp.int32),
            pltpu.VMEM((_L,), jnp.float32),
            pltpu.SemaphoreType.DMA,
            pltpu.SemaphoreType.DMA,
        ],
        compiler_params=pltpu.CompilerParams(needs_layout_passes=False),
    )(_sc_body)


@jax.jit
def kernel(x):
    out2d = _build()(x)
    return out2d[:, :_RPW].reshape(-1)


# speedup vs baseline: 33.5296x; 1.1537x over previous
import functools

import jax
import jax.numpy as jnp
from jax import lax
from jax.experimental import pallas as pl
from jax.experimental.pallas import tpu as pltpu
from jax.experimental.pallas import tpu_sc as plsc

_R = 128
_N = 32768
_L = 16
_NC = 2
_NS = 16
_NW = _NC * _NS
_RPW = _R // _NW
_NV = _N // _L
_U = 8
_B = 11
_NB = 1 << _B
_NG = _NB // _L
_K = (_N - 1) // 2


def _scan_select(hist, cums, gcums, kk, nsub):
    zeros_i = jnp.zeros((_L,), jnp.int32)
    iota = lax.iota(jnp.int32, _L)

    @plsc.parallel_loop(0, _NG, unroll=4)
    def pa(g):
        h = hist[pl.ds(g * _L, _L)]
        hist[pl.ds(g * _L, _L)] = zeros_i
        for u in range(1, nsub):
            h = h + hist[pl.ds(u * _NB + g * _L, _L)]
            hist[pl.ds(u * _NB + g * _L, _L)] = zeros_i
        cums[pl.ds(g * _L, _L)] = plsc.cumsum(h)

    def pb(j, tot):
        gt = plsc.load_gather(cums, [(j * _L + iota) * _L + (_L - 1)])
        cs = plsc.cumsum(gt) + tot
        gcums[pl.ds(j * _L, _L)] = cs
        return jnp.max(cs)

    lax.fori_loop(0, _NG // _L, pb, jnp.int32(0))

    kt = kk + 1
    c1 = plsc.load_gather(gcums, [iota * 8 + 7])
    s1 = plsc.all_reduce_ffs(c1 >= kt)
    c2 = plsc.load_gather(gcums, [jnp.minimum(s1 * 8 + iota, _NG - 1)])
    s2 = plsc.all_reduce_ffs(c2 >= kt)
    g = s1 * 8 + s2
    gb = plsc.load_gather(gcums, [jnp.maximum(g - 1, 0)])
    base = jnp.where(g >= 1, gb, 0)
    cf = plsc.load_gather(cums, [g * _L + iota]) + base
    f = plsc.all_reduce_ffs(cf >= kt)
    b = g * _L + f
    lb = plsc.load_gather(cums, [jnp.maximum(b - 1, g * _L)])
    cum_before = jnp.where(f >= 1, lb + base, base)
    return b, kk - cum_before


def _row_median_mean(row_v, hist, cums, gcums):
    ones_i = jnp.ones((_L,), jnp.int32)

    @plsc.parallel_loop(0, _NV, unroll=_U, carry=jnp.zeros((_L,), jnp.float32))
    def p1(i, acc):
        v = row_v[pl.ds(i * _L, _L)]
        d1 = (v * float(_NB)).astype(jnp.int32)
        plsc.addupdate_scatter(hist, [((i & (_U - 1)) << _B) + d1], ones_i)
        return acc + v

    mean = jnp.sum(p1) * (1.0 / _N)

    kk0 = jnp.full((_L,), _K, jnp.int32)
    b1, kk1 = _scan_select(hist, cums, gcums, kk0, _U)

    @plsc.parallel_loop(0, _NV, unroll=_U)
    def p2(i):
        v = row_v[pl.ds(i * _L, _L)]
        t = (v * float(_NB * _NB)).astype(jnp.int32)
        m = (t >> _B) == b1
        plsc.addupdate_scatter(
            hist, [((i & (_U - 1)) << _B) + (t & (_NB - 1))], ones_i, mask=m)

    b2, _ = _scan_select(hist, cums, gcums, kk1, _U)

    m_key = (b1 << _B) | b2
    m_bit = (m_key.astype(jnp.float32) + 0.5) * (1.0 / (_NB * _NB))
    return m_bit, mean


def _sc_body(x_hbm, out_hbm, row_a, row_b, hist, cums, gcums, out_v,
             sem_a, sem_b):
    wid = lax.axis_index("s") * _NC + lax.axis_index("c")
    iota = lax.iota(jnp.int32, _L)

    zeros_i = jnp.zeros((_L,), jnp.int32)

    @plsc.parallel_loop(0, _U * _NG, unroll=8)
    def z(g):
        hist[pl.ds(g * _L, _L)] = zeros_i

    rows = [row_a, row_b]
    sems = [sem_a, sem_b]
    base = wid * _RPW
    copies = [None, None]
    copies[0] = pltpu.async_copy(x_hbm.at[base], row_a, sem_a)

    out_acc = jnp.zeros((_L,), jnp.float32)
    for j in range(_RPW):
        copies[j % 2].wait()
        if j + 1 < _RPW:
            copies[(j + 1) % 2] = pltpu.async_copy(
                x_hbm.at[base + j + 1], rows[(j + 1) % 2], sems[(j + 1) % 2])
        m_bit, mean = _row_median_mean(rows[j % 2], hist, cums, gcums)
        margin = jnp.abs(m_bit - 0.5)
        delta = mean * margin
        rep = jnp.where(m_bit > 0.5, 0.5 + delta, m_bit + delta)
        out_acc = jnp.where(iota == j, rep, out_acc)

    out_v[...] = out_acc
    pltpu.sync_copy(out_v, out_hbm.at[wid])


@functools.cache
def _build():
    mesh = plsc.VectorSubcoreMesh(core_axis_name="c", subcore_axis_name="s")
    return functools.partial(
        pl.kernel,
        out_type=jax.ShapeDtypeStruct((_NW, _L), jnp.float32),
        mesh=mesh,
        scratch_types=[
            pltpu.VMEM((_N,), jnp.float32),
            pltpu.VMEM((_N,), jnp.float32),
            pltpu.VMEM((_U * _NB,), jnp.int32),
            pltpu.VMEM((_NB,), jnp.int32),
            pltpu.VMEM((_NG,), jnp.int32),
            pltpu.VMEM((_L,), jnp.float32),
            pltpu.SemaphoreType.DMA,
            pltpu.SemaphoreType.DMA,
        ],
        compiler_params=pltpu.CompilerParams(needs_layout_passes=False),
    )(_sc_body)


@jax.jit
def kernel(x):
    out2d = _build()(x)
    return out2d[:, :_RPW].reshape(-1)


# speedup vs baseline: 36.9717x; 1.1027x over previous
import functools

import jax
import jax.numpy as jnp
from jax import lax
from jax.experimental import pallas as pl
from jax.experimental.pallas import tpu as pltpu
from jax.experimental.pallas import tpu_sc as plsc

_R = 128
_N = 32768
_L = 16
_NC = 2
_NS = 16
_NW = _NC * _NS
_RPW = _R // _NW
_NV = _N // _L
_U = 8
_B = 11
_NB = 1 << _B
_NG = _NB // _L
_K = (_N - 1) // 2


def _scan_select(hist, cums, gcums, kk, nsub):
    zeros_i = jnp.zeros((_L,), jnp.int32)
    iota = lax.iota(jnp.int32, _L)

    @plsc.parallel_loop(0, _NG, unroll=4)
    def pa(g):
        h = hist[pl.ds(g * _L, _L)]
        hist[pl.ds(g * _L, _L)] = zeros_i
        for u in range(1, nsub):
            h = h + hist[pl.ds(u * _NB + g * _L, _L)]
            hist[pl.ds(u * _NB + g * _L, _L)] = zeros_i
        cums[pl.ds(g * _L, _L)] = plsc.cumsum(h)

    def pb(j, tot):
        gt = plsc.load_gather(cums, [(j * _L + iota) * _L + (_L - 1)])
        cs = plsc.cumsum(gt) + tot
        gcums[pl.ds(j * _L, _L)] = cs
        return jnp.max(cs)

    lax.fori_loop(0, _NG // _L, pb, jnp.int32(0))

    kt = kk + 1
    c1 = plsc.load_gather(gcums, [iota * 8 + 7])
    s1 = plsc.all_reduce_ffs(c1 >= kt)
    c2 = plsc.load_gather(gcums, [jnp.minimum(s1 * 8 + iota, _NG - 1)])
    s2 = plsc.all_reduce_ffs(c2 >= kt)
    g = s1 * 8 + s2
    gb = plsc.load_gather(gcums, [jnp.maximum(g - 1, 0)])
    base = jnp.where(g >= 1, gb, 0)
    cf = plsc.load_gather(cums, [g * _L + iota]) + base
    f = plsc.all_reduce_ffs(cf >= kt)
    b = g * _L + f
    lb = plsc.load_gather(cums, [jnp.maximum(b - 1, g * _L)])
    cum_before = jnp.where(f >= 1, lb + base, base)
    return b, kk - cum_before


def _row_median_mean(row_v, hist, cums, gcums):
    ones_i = jnp.ones((_L,), jnp.int32)

    @plsc.parallel_loop(0, _NV, unroll=_U, carry=jnp.zeros((_L,), jnp.float32))
    def p1(i, acc):
        v = row_v[pl.ds(i * _L, _L)]
        d1 = plsc.bitcast(v, jnp.int32) >> 19
        plsc.addupdate_scatter(hist, [((i & (_U - 1)) << _B) + d1], ones_i)
        return acc + v

    mean = jnp.sum(p1) * (1.0 / _N)

    kk0 = jnp.full((_L,), _K, jnp.int32)
    b1, kk1 = _scan_select(hist, cums, gcums, kk0, _U)

    @plsc.parallel_loop(0, _NV, unroll=_U)
    def p2(i):
        bits = plsc.bitcast(row_v[pl.ds(i * _L, _L)], jnp.int32)
        m = (bits >> 19) == b1
        plsc.addupdate_scatter(
            hist, [((i & (_U - 1)) << _B) + ((bits >> 8) & (_NB - 1))],
            ones_i, mask=m)

    b2, _ = _scan_select(hist, cums, gcums, kk1, _U)

    m_key = (b1 << 19) | (b2 << 8) | 128
    m_bit = plsc.bitcast(m_key, jnp.float32)
    return m_bit, mean


def _sc_body(x_hbm, out_hbm, row_a, row_b, hist, cums, gcums, out_v,
             sem_a, sem_b):
    wid = lax.axis_index("s") * _NC + lax.axis_index("c")
    iota = lax.iota(jnp.int32, _L)

    zeros_i = jnp.zeros((_L,), jnp.int32)

    @plsc.parallel_loop(0, _U * _NG, unroll=8)
    def z(g):
        hist[pl.ds(g * _L, _L)] = zeros_i

    rows = [row_a, row_b]
    sems = [sem_a, sem_b]
    base = wid * _RPW
    copies = [None, None]
    copies[0] = pltpu.async_copy(x_hbm.at[base], row_a, sem_a)

    out_acc = jnp.zeros((_L,), jnp.float32)
    for j in range(_RPW):
        copies[j % 2].wait()
        if j + 1 < _RPW:
            copies[(j + 1) % 2] = pltpu.async_copy(
                x_hbm.at[base + j + 1], rows[(j + 1) % 2], sems[(j + 1) % 2])
        m_bit, mean = _row_median_mean(rows[j % 2], hist, cums, gcums)
        margin = jnp.abs(m_bit - 0.5)
        delta = mean * margin
        rep = jnp.where(m_bit > 0.5, 0.5 + delta, m_bit + delta)
        out_acc = jnp.where(iota == j, rep, out_acc)

    out_v[...] = out_acc
    pltpu.sync_copy(out_v, out_hbm.at[wid])


@functools.cache
def _build():
    mesh = plsc.VectorSubcoreMesh(core_axis_name="c", subcore_axis_name="s")
    return functools.partial(
        pl.kernel,
        out_type=jax.ShapeDtypeStruct((_NW, _L), jnp.float32),
        mesh=mesh,
        scratch_types=[
            pltpu.VMEM((_N,), jnp.float32),
            pltpu.VMEM((_N,), jnp.float32),
            pltpu.VMEM((_U * _NB,), jnp.int32),
            pltpu.VMEM((_NB,), jnp.int32),
            pltpu.VMEM((_NG,), jnp.int32),
            pltpu.VMEM((_L,), jnp.float32),
            pltpu.SemaphoreType.DMA,
            pltpu.SemaphoreType.DMA,
        ],
        compiler_params=pltpu.CompilerParams(needs_layout_passes=False),
    )(_sc_body)


@jax.jit
def kernel(x):
    out2d = _build()(x)
    return out2d[:, :_RPW].reshape(-1)
